# baseline (speedup 1.0000x reference)
import functools
import jax
import jax.numpy as jnp
from jax import lax
from jax.experimental import pallas as pl
from jax.experimental.pallas import tpu as pltpu
from jax.experimental.pallas import tpu_sc as plsc


def kernel(x, table):
    B = x.shape[0]
    T, D = table.shape
    info = plsc.get_sparse_core_info()
    NW = info.num_cores * info.num_subcores
    rows_per_w = T // NW
    CH = 16
    nch = rows_per_w // CH

    mesh = plsc.VectorSubcoreMesh(core_axis_name="c", subcore_axis_name="s")

    @functools.partial(
        pl.kernel,
        mesh=mesh,
        out_type=jax.ShapeDtypeStruct((B, T, D), jnp.float32),
        scratch_types=[
            pltpu.VMEM((CH, D), jnp.float32),
            pltpu.VMEM((CH, D), jnp.float32),
            pltpu.SemaphoreType.DMA,
            pltpu.SemaphoreType.DMA,
            pltpu.SemaphoreType.DMA,
            pltpu.SemaphoreType.DMA,
        ],
    )
    def k(table_hbm, out_hbm, buf0, buf1, rsem0, rsem1, wsem0, wsem1):
        wid = lax.axis_index("s") * info.num_cores + lax.axis_index("c")
        base = wid * rows_per_w
        bufs = (buf0, buf1)
        rsems = (rsem0, rsem1)
        wsems = (wsem0, wsem1)

        pltpu.make_async_copy(table_hbm.at[pl.ds(base, CH)], buf0, rsem0).start()

        for c in range(nch):
            s = c % 2
            ns = (c + 1) % 2
            buf, rsem, wsem = bufs[s], rsems[s], wsems[s]
            r0 = base + c * CH
            pltpu.make_async_copy(table_hbm.at[pl.ds(r0, CH)], buf, rsem).wait()
            if c + 1 < nch:
                if c >= 1:
                    pr0 = base + (c - 1) * CH
                    for b in range(B):
                        pltpu.make_async_copy(
                            bufs[ns], out_hbm.at[b, pl.ds(pr0, CH)], wsems[ns]
                        ).wait()
                nr0 = base + (c + 1) * CH
                pltpu.make_async_copy(
                    table_hbm.at[pl.ds(nr0, CH)], bufs[ns], rsems[ns]
                ).start()
            for b in range(B):
                pltpu.make_async_copy(
                    buf, out_hbm.at[b, pl.ds(r0, CH)], wsem
                ).start()

        for c in (nch - 2, nch - 1):
            s = c % 2
            r0 = base + c * CH
            for b in range(B):
                pltpu.make_async_copy(
                    bufs[s], out_hbm.at[b, pl.ds(r0, CH)], wsems[s]
                ).wait()

    return k(table)


# speedup vs baseline: 1.1041x; 1.1041x over previous
import functools
import jax
import jax.numpy as jnp
from jax import lax
from jax.experimental import pallas as pl
from jax.experimental.pallas import tpu as pltpu
from jax.experimental.pallas import tpu_sc as plsc


def kernel(x, table):
    B = x.shape[0]
    T, D = table.shape
    info = plsc.get_sparse_core_info()
    NW = info.num_cores * info.num_subcores
    rows_per_w = T // NW
    CH = 56
    sizes = [56, 56, 56, 56, 32]
    offs = [0, 56, 112, 168, 224]
    nch = len(sizes)

    mesh = plsc.VectorSubcoreMesh(core_axis_name="c", subcore_axis_name="s")

    @functools.partial(
        pl.kernel,
        mesh=mesh,
        out_type=jax.ShapeDtypeStruct((B, T, D), jnp.float32),
        scratch_types=[
            pltpu.VMEM((CH, D), jnp.float32),
            pltpu.VMEM((CH, D), jnp.float32),
            pltpu.SemaphoreType.DMA,
            pltpu.SemaphoreType.DMA,
            pltpu.SemaphoreType.DMA,
            pltpu.SemaphoreType.DMA,
        ],
    )
    def k(table_hbm, out_hbm, buf0, buf1, rsem0, rsem1, wsem0, wsem1):
        wid = lax.axis_index("s") * info.num_cores + lax.axis_index("c")
        base = wid * rows_per_w
        bufs = (buf0, buf1)
        rsems = (rsem0, rsem1)
        wsems = (wsem0, wsem1)

        def rd(c, s):
            return pltpu.make_async_copy(
                table_hbm.at[pl.ds(base + offs[c], sizes[c])],
                bufs[s].at[pl.ds(0, sizes[c])],
                rsems[s],
            )

        def wr(c, s, b):
            return pltpu.make_async_copy(
                bufs[s].at[pl.ds(0, sizes[c])],
                out_hbm.at[b, pl.ds(base + offs[c], sizes[c])],
                wsems[s],
            )

        rd(0, 0).start()

        for c in range(nch):
            s = c % 2
            ns = (c + 1) % 2
            rd(c, s).wait()
            if c + 1 < nch:
                if c >= 1:
                    for b in range(B):
                        wr(c - 1, ns, b).wait()
                rd(c + 1, ns).start()
            for b in range(B):
                wr(c, s, b).start()

        for c in (nch - 2, nch - 1):
            for b in range(B):
                wr(c, c % 2, b).wait()

    return k(table)
